# Initial kernel scaffold; baseline (speedup 1.0000x reference)
import jax
import jax.numpy as jnp
from jax.experimental import pallas as pl


def kernel(symbol_embeddings, question_symbols, ranking_difference, segment_ids, W, b):
    raise NotImplementedError("write your pallas kernel here")


# speedup vs baseline: 85.3768x; 85.3768x over previous
import functools

import jax
import jax.numpy as jnp
from jax import lax
from jax.experimental import pallas as pl
from jax.experimental.pallas import tpu as pltpu
from jax.experimental.pallas import tpu_sc as plsc

N_SYM = 100000
M = 3200000
D = 64
NUM_SEGMENTS = 100000

NW = 32
CHUNK = M // NW
SUB = 2000
N_STEPS = CHUNK // SUB
BATCH = 80
ROWS = SUB // BATCH
VPS = SUB // 16
N_PAD = 102400
SLICE = N_PAD // 16


def _costs_body(emb_ref, w_ref, b_ref, out_ref):
    out_ref[...] = jnp.dot(
        emb_ref[...], w_ref[...], preferred_element_type=jnp.float32
    ) + b_ref[0, 0]


def _symbol_costs(emb, W, b):
    grid = 10
    rows = N_SYM // grid
    return pl.pallas_call(
        _costs_body,
        grid=(grid,),
        in_specs=[
            pl.BlockSpec((rows, D), lambda i: (i, 0)),
            pl.BlockSpec((D, 1), lambda i: (0, 0)),
            pl.BlockSpec((1, 1), lambda i: (0, 0)),
        ],
        out_specs=pl.BlockSpec((rows, 1), lambda i: (i, 0)),
        out_shape=jax.ShapeDtypeStruct((N_SYM, 1), jnp.float32),
    )(emb, W, b.reshape(1, 1))


def _sc_body(costs_hbm, qs_hbm, rd_hbm, sid_hbm, out_hbm,
             costs_l, qs_s, rd_s, pot_s, sid_s, zbuf, acc):
    c = lax.axis_index("c")
    s = lax.axis_index("s")
    w = c * 16 + s

    def _z(i, carry):
        zbuf[pl.ds(pl.multiple_of(i * 16, 16), 16)] = jnp.zeros(
            (16,), jnp.float32)
        return carry
    lax.fori_loop(0, SLICE // 16, _z, 0)
    pltpu.sync_copy(zbuf, acc.at[pl.ds(s * SLICE, SLICE)])

    pltpu.sync_copy(costs_hbm, costs_l)
    plsc.subcore_barrier()

    def _step(g, carry):
        base = w * CHUNK + g * SUB
        row = (w * CHUNK + g * SUB) // BATCH
        pltpu.sync_copy(qs_hbm.at[pl.ds(base, SUB)], qs_s)
        pltpu.sync_copy(rd_hbm.at[pl.ds(base, SUB)], rd_s)
        pltpu.sync_copy(sid_hbm.at[pl.ds(row, ROWS)], sid_s)

        def _vec(i, carry2):
            off = pl.ds(pl.multiple_of(i * 16, 16), 16)
            qv = qs_s[off]
            rv = rd_s[off]
            cv = plsc.load_gather(costs_l, [qv])
            pot_s[off] = cv * rv
            return carry2
        lax.fori_loop(0, VPS, _vec, 0)

        def _scat(j, carry2):
            pltpu.sync_copy(
                pot_s.at[pl.ds(pl.multiple_of(j * BATCH, 8), BATCH)],
                acc.at[sid_s.at[j]],
                add=True,
            )
            return carry2
        lax.fori_loop(0, ROWS, _scat, 0)
        return carry

    lax.fori_loop(0, N_STEPS, _step, 0)

    plsc.subcore_barrier()
    pltpu.sync_copy(
        acc.at[pl.ds(s * SLICE, SLICE)],
        out_hbm.at[c, pl.ds(s * SLICE, SLICE)],
    )


def _sc_call(costs, qs, rd, sid2):
    mesh = plsc.VectorSubcoreMesh(core_axis_name="c", subcore_axis_name="s")
    f = pl.kernel(
        _sc_body,
        out_type=jax.ShapeDtypeStruct((2, N_PAD), jnp.float32),
        mesh=mesh,
        scratch_types=[
            pltpu.VMEM((N_SYM,), jnp.float32),
            pltpu.VMEM((SUB,), jnp.int32),
            pltpu.VMEM((SUB,), jnp.float32),
            pltpu.VMEM((SUB,), jnp.float32),
            pltpu.VMEM((ROWS, BATCH), jnp.int32),
            pltpu.VMEM((SLICE,), jnp.float32),
            pltpu.VMEM_SHARED((N_PAD,), jnp.float32),
        ],
        compiler_params=pltpu.CompilerParams(
            use_tc_tiling_on_sc=False, needs_layout_passes=False),
    )
    return f(costs, qs, rd, sid2)


def _merge_body(p_ref, out_ref):
    out_ref[...] = p_ref[0] + p_ref[1]


def _merge(partials):
    p3 = partials.reshape(2, N_PAD // 128, 128)
    return pl.pallas_call(
        _merge_body,
        out_shape=jax.ShapeDtypeStruct((N_PAD // 128, 128), jnp.float32),
    )(p3)


def kernel(symbol_embeddings, question_symbols, ranking_difference,
           segment_ids, W, b):
    costs = _symbol_costs(symbol_embeddings, W, b).reshape(N_SYM)
    qs = question_symbols.astype(jnp.int32)
    sid2 = segment_ids.astype(jnp.int32).reshape(M // BATCH, BATCH)
    partials = _sc_call(costs, qs, ranking_difference, sid2)
    out = _merge(partials).reshape(N_PAD)
    return out[:NUM_SEGMENTS]


# speedup vs baseline: 103.5965x; 1.2134x over previous
import jax
import jax.numpy as jnp
from jax import lax
from jax.experimental import pallas as pl
from jax.experimental.pallas import tpu as pltpu
from jax.experimental.pallas import tpu_sc as plsc

N_SYM = 100000
M = 3200000
D = 64
NUM_SEGMENTS = 100000

NW = 32
CHUNK = M // NW
SUB = 2000
STEPS = CHUNK // SUB
VPS = SUB // 16
WIN = 8192
FB = 128
N_PAD = 102400
SLICE = N_PAD // 16
NEG = -(2 ** 30)
POS = 2 ** 30


def _costs_body(emb_ref, w_ref, b_ref, out_ref):
    out_ref[...] = jnp.dot(
        emb_ref[...], w_ref[...], preferred_element_type=jnp.float32
    ) + b_ref[0, 0]


def _symbol_costs(emb, W, b):
    grid = 10
    rows = N_SYM // grid
    return pl.pallas_call(
        _costs_body,
        grid=(grid,),
        in_specs=[
            pl.BlockSpec((rows, D), lambda i: (i, 0)),
            pl.BlockSpec((D, 1), lambda i: (0, 0)),
            pl.BlockSpec((1, 1), lambda i: (0, 0)),
        ],
        out_specs=pl.BlockSpec((rows, 1), lambda i: (i, 0)),
        out_shape=jax.ShapeDtypeStruct((N_SYM, 1), jnp.float32),
    )(emb, W, b.reshape(1, 1))


def _sc_body(costs_hbm, qs_hbm, rd_hbm, sid_hbm, out_hbm,
             costs_l, qs_s, rd_s, sid_s, win, fidx, spacc, sems, csem):
    c = lax.axis_index("c")
    s = lax.axis_index("s")
    w = c * 16 + s
    lanes = lax.iota(jnp.int32, 16)
    zero16 = jnp.zeros((16,), jnp.float32)

    def flush(base, maxs):
        n = jnp.clip(maxs + 1 - base, 0, WIN)
        trips = (n + FB - 1) // FB

        def fb(j, carry):
            joff = pl.multiple_of(j * FB, FB)
            for k in range(FB // 16):
                fidx[pl.ds(k * 16, 16)] = jnp.clip(
                    base + j * FB + k * 16 + lanes, 0, N_PAD - 1)
            pltpu.sync_copy(win.at[pl.ds(joff, FB)],
                            spacc.at[fidx], add=True)
            for k in range(FB // 16):
                win[pl.ds(joff + k * 16, 16)] = zero16
            return carry
        lax.fori_loop(0, trips, fb, 0)

    def zb(i, carry):
        win[pl.ds(pl.multiple_of(i * 16, 16), 16)] = zero16
        return carry
    lax.fori_loop(0, WIN // 16, zb, 0)

    pltpu.async_copy(costs_hbm, costs_l, csem)
    base0 = w * CHUNK
    pltpu.async_copy(qs_hbm.at[pl.ds(base0, SUB)], qs_s.at[0], sems.at[0])
    pltpu.async_copy(rd_hbm.at[pl.ds(base0, SUB)], rd_s.at[0], sems.at[0])
    pltpu.async_copy(sid_hbm.at[pl.ds(base0, SUB)], sid_s.at[0], sems.at[0])

    pltpu.sync_copy(win.at[pl.ds(0, SLICE)],
                    spacc.at[pl.ds(s * SLICE, SLICE)])
    pltpu.make_async_copy(costs_hbm, costs_l, csem).wait()
    plsc.subcore_barrier()

    def step(g, carry, slot, nslot):
        base, maxs = carry

        @pl.when(g < STEPS - 1)
        def _():
            nb = w * CHUNK + (g + 1) * SUB
            pltpu.async_copy(qs_hbm.at[pl.ds(nb, SUB)], qs_s.at[nslot],
                             sems.at[nslot])
            pltpu.async_copy(rd_hbm.at[pl.ds(nb, SUB)], rd_s.at[nslot],
                             sems.at[nslot])
            pltpu.async_copy(sid_hbm.at[pl.ds(nb, SUB)], sid_s.at[nslot],
                             sems.at[nslot])

        gb = w * CHUNK + g * SUB
        pltpu.make_async_copy(
            qs_hbm.at[pl.ds(gb, SUB)], qs_s.at[slot], sems.at[slot]).wait()
        pltpu.make_async_copy(
            rd_hbm.at[pl.ds(gb, SUB)], rd_s.at[slot], sems.at[slot]).wait()
        pltpu.make_async_copy(
            sid_hbm.at[pl.ds(gb, SUB)], sid_s.at[slot], sems.at[slot]).wait()

        qs_b = qs_s.at[slot]
        rd_b = rd_s.at[slot]
        sid_b = sid_s.at[slot]

        def vm(i, acc):
            return jnp.maximum(acc, sid_b[pl.ds(pl.multiple_of(i * 16, 16),
                                                16)])
        cmax = jnp.max(lax.fori_loop(0, VPS, vm,
                                     jnp.full((16,), NEG, jnp.int32)))

        def fast(carry):
            base, maxs = carry

            @pl.loop(0, VPS)
            def _(i):
                off = pl.ds(pl.multiple_of(i * 16, 16), 16)
                sid_v = sid_b[off]
                cv = plsc.load_gather(costs_l, [qs_b[off]])
                plsc.addupdate_scatter(win, [sid_v - base], cv * rd_b[off])
            return base, jnp.maximum(maxs, cmax)

        def slow(carry):
            def attempt(b, m, r, sid_v, pot):
                loc = sid_v - b
                m_in = (loc < WIN) & (r == 1)
                loc = jnp.clip(loc, 0, WIN - 1)
                plsc.addupdate_scatter(win, [loc], pot, mask=m_in)
                m2 = jnp.maximum(m, jnp.max(jnp.where(m_in, sid_v, m)))
                r2 = jnp.where(m_in, 0, r)
                return b, m2, r2

            def sv(i, carry2):
                base, maxs = carry2
                off = pl.ds(pl.multiple_of(i * 16, 16), 16)
                sid_v = sid_b[off]
                pot = plsc.load_gather(costs_l, [qs_b[off]]) * rd_b[off]
                ones = jnp.ones((16,), jnp.int32)
                b, m, r = attempt(base, maxs, ones, sid_v, pot)

                def wcond(cr):
                    return jnp.max(cr[2]) > 0

                def wbody(cr):
                    b, m, r = cr
                    flush(b, m)
                    b2 = jnp.min(jnp.where(r == 1, sid_v, POS))
                    return attempt(b2, b2 - 1, r, sid_v, pot)

                b, m, r = lax.while_loop(wcond, wbody, (b, m, r))
                return b, m
            return lax.fori_loop(0, VPS, sv, carry)

        return lax.cond(cmax < base + WIN, fast, slow, (base, maxs))

    def pair(t, carry):
        g = t * 2
        carry = step(g, carry, 0, 1)
        carry = step(g + 1, carry, 1, 0)
        return carry

    base, maxs = lax.fori_loop(
        0, STEPS // 2, pair, (jnp.int32(NEG), jnp.int32(NEG - 1)))
    flush(base, maxs)
    plsc.subcore_barrier()
    pltpu.sync_copy(
        spacc.at[pl.ds(s * SLICE, SLICE)],
        out_hbm.at[c, pl.ds(s * SLICE, SLICE)],
    )


def _sc_call(costs, qs, rd, sid):
    mesh = plsc.VectorSubcoreMesh(core_axis_name="c", subcore_axis_name="s")
    f = pl.kernel(
        _sc_body,
        out_type=jax.ShapeDtypeStruct((2, N_PAD), jnp.float32),
        mesh=mesh,
        scratch_types=[
            pltpu.VMEM((N_SYM,), jnp.float32),
            pltpu.VMEM((2, SUB), jnp.int32),
            pltpu.VMEM((2, SUB), jnp.float32),
            pltpu.VMEM((2, SUB), jnp.int32),
            pltpu.VMEM((WIN,), jnp.float32),
            pltpu.VMEM((FB,), jnp.int32),
            pltpu.VMEM_SHARED((N_PAD,), jnp.float32),
            pltpu.SemaphoreType.DMA((2,)),
            pltpu.SemaphoreType.DMA,
        ],
        compiler_params=pltpu.CompilerParams(
            use_tc_tiling_on_sc=False, needs_layout_passes=False),
    )
    return f(costs, qs, rd, sid)


def _merge_body(p_ref, out_ref):
    out_ref[...] = p_ref[0] + p_ref[1]


def _merge(partials):
    p3 = partials.reshape(2, N_PAD // 128, 128)
    return pl.pallas_call(
        _merge_body,
        out_shape=jax.ShapeDtypeStruct((N_PAD // 128, 128), jnp.float32),
    )(p3)


def kernel(symbol_embeddings, question_symbols, ranking_difference,
           segment_ids, W, b):
    costs = _symbol_costs(symbol_embeddings, W, b).reshape(N_SYM)
    qs = question_symbols.astype(jnp.int32)
    sid = segment_ids.astype(jnp.int32)
    partials = _sc_call(costs, qs, ranking_difference, sid)
    out = _merge(partials).reshape(N_PAD)
    return out[:NUM_SEGMENTS]


# speedup vs baseline: 113.7639x; 1.0981x over previous
import jax
import jax.numpy as jnp
from jax import lax
from jax.experimental import pallas as pl
from jax.experimental.pallas import tpu as pltpu
from jax.experimental.pallas import tpu_sc as plsc

N_SYM = 100000
M = 3200000
D = 64
NUM_SEGMENTS = 100000

NW = 32
CHUNK = M // NW
SUB = 2000
STEPS = CHUNK // SUB
VPS = SUB // 16
WIN = 8192
FB = 128
N_PAD = 102400
SLICE = N_PAD // 16
NEG = -(2 ** 30)
POS = 2 ** 30


def _costs_body(emb_ref, w_ref, b_ref, out_ref):
    out_ref[...] = jnp.dot(
        emb_ref[...], w_ref[...], preferred_element_type=jnp.float32
    ) + b_ref[0, 0]


def _symbol_costs(emb, W, b):
    grid = 10
    rows = N_SYM // grid
    return pl.pallas_call(
        _costs_body,
        grid=(grid,),
        in_specs=[
            pl.BlockSpec((rows, D), lambda i: (i, 0)),
            pl.BlockSpec((D, 1), lambda i: (0, 0)),
            pl.BlockSpec((1, 1), lambda i: (0, 0)),
        ],
        out_specs=pl.BlockSpec((rows, 1), lambda i: (i, 0)),
        out_shape=jax.ShapeDtypeStruct((N_SYM, 1), jnp.float32),
    )(emb, W, b.reshape(1, 1))


def _sc_body(costs_hbm, qs_hbm, rd_hbm, sid_hbm, out_hbm,
             costs_l, qs_s, rd_s, sid_s, win, fidx, spacc, sems, csem):
    c = lax.axis_index("c")
    s = lax.axis_index("s")
    w = c * 16 + s
    lanes = lax.iota(jnp.int32, 16)
    zero16 = jnp.zeros((16,), jnp.float32)

    def flush(base, maxs):
        n = jnp.clip(maxs + 1 - base, 0, WIN)
        trips = (n + FB - 1) // FB

        def fb(j, carry):
            joff = pl.multiple_of(j * FB, FB)
            for k in range(FB // 16):
                fidx[pl.ds(k * 16, 16)] = jnp.clip(
                    base + j * FB + k * 16 + lanes, 0, N_PAD - 1)
            pltpu.sync_copy(win.at[pl.ds(joff, FB)],
                            spacc.at[fidx], add=True)
            for k in range(FB // 16):
                win[pl.ds(joff + k * 16, 16)] = zero16
            return carry
        lax.fori_loop(0, trips, fb, 0)

    def zb(i, carry):
        win[pl.ds(pl.multiple_of(i * 16, 16), 16)] = zero16
        return carry
    lax.fori_loop(0, WIN // 16, zb, 0)

    pltpu.async_copy(costs_hbm, costs_l, csem)
    base0 = w * CHUNK
    pltpu.async_copy(qs_hbm.at[pl.ds(base0, SUB)], qs_s.at[0], sems.at[0])
    pltpu.async_copy(rd_hbm.at[pl.ds(base0, SUB)], rd_s.at[0], sems.at[0])
    pltpu.async_copy(sid_hbm.at[pl.ds(base0, SUB)], sid_s.at[0], sems.at[0])

    pltpu.sync_copy(win.at[pl.ds(0, SLICE)],
                    spacc.at[pl.ds(s * SLICE, SLICE)])
    pltpu.make_async_copy(costs_hbm, costs_l, csem).wait()
    plsc.subcore_barrier()

    def step(g, carry, slot, nslot):
        base, maxs = carry

        @pl.when(g < STEPS - 1)
        def _():
            nb = w * CHUNK + (g + 1) * SUB
            pltpu.async_copy(qs_hbm.at[pl.ds(nb, SUB)], qs_s.at[nslot],
                             sems.at[nslot])
            pltpu.async_copy(rd_hbm.at[pl.ds(nb, SUB)], rd_s.at[nslot],
                             sems.at[nslot])
            pltpu.async_copy(sid_hbm.at[pl.ds(nb, SUB)], sid_s.at[nslot],
                             sems.at[nslot])

        gb = w * CHUNK + g * SUB
        pltpu.make_async_copy(
            qs_hbm.at[pl.ds(gb, SUB)], qs_s.at[slot], sems.at[slot]).wait()
        pltpu.make_async_copy(
            rd_hbm.at[pl.ds(gb, SUB)], rd_s.at[slot], sems.at[slot]).wait()
        pltpu.make_async_copy(
            sid_hbm.at[pl.ds(gb, SUB)], sid_s.at[slot], sems.at[slot]).wait()

        qs_b = qs_s.at[slot]
        rd_b = rd_s.at[slot]
        sid_b = sid_s.at[slot]

        cmax = jnp.max(sid_b[pl.ds(SUB - 16, 16)])

        def fast(carry):
            base, maxs = carry

            @pl.loop(0, VPS, unroll=5)
            def _(i):
                off = pl.ds(pl.multiple_of(i * 16, 16), 16)
                sid_v = sid_b[off]
                cv = plsc.load_gather(costs_l, [qs_b[off]])
                plsc.addupdate_scatter(win, [sid_v - base], cv * rd_b[off])
            return base, jnp.maximum(maxs, cmax)

        def slow(carry):
            def attempt(b, m, r, sid_v, pot):
                loc = sid_v - b
                m_in = (loc < WIN) & (r == 1)
                loc = jnp.clip(loc, 0, WIN - 1)
                plsc.addupdate_scatter(win, [loc], pot, mask=m_in)
                m2 = jnp.maximum(m, jnp.max(jnp.where(m_in, sid_v, m)))
                r2 = jnp.where(m_in, 0, r)
                return b, m2, r2

            def sv(i, carry2):
                base, maxs = carry2
                off = pl.ds(pl.multiple_of(i * 16, 16), 16)
                sid_v = sid_b[off]
                pot = plsc.load_gather(costs_l, [qs_b[off]]) * rd_b[off]
                ones = jnp.ones((16,), jnp.int32)
                b, m, r = attempt(base, maxs, ones, sid_v, pot)

                def wcond(cr):
                    return jnp.max(cr[2]) > 0

                def wbody(cr):
                    b, m, r = cr
                    flush(b, m)
                    b2 = jnp.min(jnp.where(r == 1, sid_v, POS))
                    return attempt(b2, b2 - 1, r, sid_v, pot)

                b, m, r = lax.while_loop(wcond, wbody, (b, m, r))
                return b, m
            return lax.fori_loop(0, VPS, sv, carry)

        return lax.cond(cmax < base + WIN, fast, slow, (base, maxs))

    def pair(t, carry):
        g = t * 2
        carry = step(g, carry, 0, 1)
        carry = step(g + 1, carry, 1, 0)
        return carry

    base, maxs = lax.fori_loop(
        0, STEPS // 2, pair, (jnp.int32(NEG), jnp.int32(NEG - 1)))
    flush(base, maxs)
    plsc.subcore_barrier()
    pltpu.sync_copy(
        spacc.at[pl.ds(s * SLICE, SLICE)],
        out_hbm.at[c, pl.ds(s * SLICE, SLICE)],
    )


def _sc_call(costs, qs, rd, sid):
    mesh = plsc.VectorSubcoreMesh(core_axis_name="c", subcore_axis_name="s")
    f = pl.kernel(
        _sc_body,
        out_type=jax.ShapeDtypeStruct((2, N_PAD), jnp.float32),
        mesh=mesh,
        scratch_types=[
            pltpu.VMEM((N_SYM,), jnp.float32),
            pltpu.VMEM((2, SUB), jnp.int32),
            pltpu.VMEM((2, SUB), jnp.float32),
            pltpu.VMEM((2, SUB), jnp.int32),
            pltpu.VMEM((WIN,), jnp.float32),
            pltpu.VMEM((FB,), jnp.int32),
            pltpu.VMEM_SHARED((N_PAD,), jnp.float32),
            pltpu.SemaphoreType.DMA((2,)),
            pltpu.SemaphoreType.DMA,
        ],
        compiler_params=pltpu.CompilerParams(
            use_tc_tiling_on_sc=False, needs_layout_passes=False),
    )
    return f(costs, qs, rd, sid)


def _merge_body(p_ref, out_ref):
    out_ref[...] = p_ref[0] + p_ref[1]


def _merge(partials):
    p3 = partials.reshape(2, N_PAD // 128, 128)
    return pl.pallas_call(
        _merge_body,
        out_shape=jax.ShapeDtypeStruct((N_PAD // 128, 128), jnp.float32),
    )(p3)


def kernel(symbol_embeddings, question_symbols, ranking_difference,
           segment_ids, W, b):
    costs = _symbol_costs(symbol_embeddings, W, b).reshape(N_SYM)
    qs = question_symbols.astype(jnp.int32)
    sid = segment_ids.astype(jnp.int32)
    partials = _sc_call(costs, qs, ranking_difference, sid)
    out = _merge(partials).reshape(N_PAD)
    return out[:NUM_SEGMENTS]


# speedup vs baseline: 136.6800x; 1.2014x over previous
import jax
import jax.numpy as jnp
from jax import lax
from jax.experimental import pallas as pl
from jax.experimental.pallas import tpu as pltpu
from jax.experimental.pallas import tpu_sc as plsc

N_SYM = 100000
M = 3200000
D = 64
NUM_SEGMENTS = 100000

NW = 32
CHUNK = M // NW
SUB = 2000
STEPS = CHUNK // SUB
VPS = SUB // 16
WIN = 8192
FB = 128
N_PAD = 102400
SLICE = N_PAD // 16
NEG = -(2 ** 30)
POS = 2 ** 30


def _costs_body(emb_ref, w_ref, b_ref, out_ref):
    out_ref[...] = jnp.dot(
        emb_ref[...], w_ref[...], preferred_element_type=jnp.float32
    ) + b_ref[0, 0]


def _symbol_costs(emb, W, b):
    grid = 10
    rows = N_SYM // grid
    return pl.pallas_call(
        _costs_body,
        grid=(grid,),
        in_specs=[
            pl.BlockSpec((rows, D), lambda i: (i, 0)),
            pl.BlockSpec((D, 1), lambda i: (0, 0)),
            pl.BlockSpec((1, 1), lambda i: (0, 0)),
        ],
        out_specs=pl.BlockSpec((rows, 1), lambda i: (i, 0)),
        out_shape=jax.ShapeDtypeStruct((N_SYM, 1), jnp.float32),
    )(emb, W, b.reshape(1, 1))


def _sc_body(costs_hbm, qs_hbm, rd_hbm, sid_hbm, out_hbm,
             costs_l, qs_s, rd_s, sid_s, win, fidx, spacc, sems, csem):
    c = lax.axis_index("c")
    s = lax.axis_index("s")
    w = c * 16 + s
    lanes = lax.iota(jnp.int32, 16)
    zero16 = jnp.zeros((16,), jnp.float32)

    def flush(base, maxs):
        n = jnp.clip(maxs + 1 - base, 0, WIN)
        trips = (n + FB - 1) // FB

        def fb(j, carry):
            joff = pl.multiple_of(j * FB, FB)
            for k in range(FB // 16):
                fidx[pl.ds(k * 16, 16)] = jnp.clip(
                    base + j * FB + k * 16 + lanes, 0, N_PAD - 1)
            pltpu.sync_copy(win.at[pl.ds(joff, FB)],
                            spacc.at[fidx], add=True)
            for k in range(FB // 16):
                win[pl.ds(joff + k * 16, 16)] = zero16
            return carry
        lax.fori_loop(0, trips, fb, 0)

    def zb(i, carry):
        win[pl.ds(pl.multiple_of(i * 16, 16), 16)] = zero16
        return carry
    lax.fori_loop(0, WIN // 16, zb, 0)

    pltpu.async_copy(costs_hbm, costs_l, csem)
    base0 = w * CHUNK
    pltpu.async_copy(qs_hbm.at[pl.ds(base0, SUB)], qs_s.at[0], sems.at[0])
    pltpu.async_copy(rd_hbm.at[pl.ds(base0, SUB)], rd_s.at[0], sems.at[0])
    pltpu.async_copy(sid_hbm.at[pl.ds(base0, SUB)], sid_s.at[0], sems.at[0])

    pltpu.sync_copy(win.at[pl.ds(0, SLICE)],
                    spacc.at[pl.ds(s * SLICE, SLICE)])
    pltpu.make_async_copy(costs_hbm, costs_l, csem).wait()
    plsc.subcore_barrier()

    def step(g, carry, slot, nslot):
        base, maxs = carry

        @pl.when(g < STEPS - 1)
        def _():
            nb = w * CHUNK + (g + 1) * SUB
            pltpu.async_copy(qs_hbm.at[pl.ds(nb, SUB)], qs_s.at[nslot],
                             sems.at[nslot])
            pltpu.async_copy(rd_hbm.at[pl.ds(nb, SUB)], rd_s.at[nslot],
                             sems.at[nslot])
            pltpu.async_copy(sid_hbm.at[pl.ds(nb, SUB)], sid_s.at[nslot],
                             sems.at[nslot])

        gb = w * CHUNK + g * SUB
        pltpu.make_async_copy(
            qs_hbm.at[pl.ds(gb, SUB)], qs_s.at[slot], sems.at[slot]).wait()
        pltpu.make_async_copy(
            rd_hbm.at[pl.ds(gb, SUB)], rd_s.at[slot], sems.at[slot]).wait()
        pltpu.make_async_copy(
            sid_hbm.at[pl.ds(gb, SUB)], sid_s.at[slot], sems.at[slot]).wait()

        qs_b = qs_s.at[slot]
        rd_b = rd_s.at[slot]
        sid_b = sid_s.at[slot]

        cmax = jnp.max(sid_b[pl.ds(SUB - 16, 16)])

        def fast(carry):
            base, maxs = carry

            @plsc.parallel_loop(0, VPS, unroll=5)
            def _(i):
                off = pl.ds(pl.multiple_of(i * 16, 16), 16)
                sid_v = sid_b[off]
                cv = plsc.load_gather(costs_l, [qs_b[off]])
                plsc.addupdate_scatter(win, [sid_v - base], cv * rd_b[off])
            return base, jnp.maximum(maxs, cmax)

        def slow(carry):
            def attempt(b, m, r, sid_v, pot):
                loc = sid_v - b
                m_in = (loc < WIN) & (r == 1)
                loc = jnp.clip(loc, 0, WIN - 1)
                plsc.addupdate_scatter(win, [loc], pot, mask=m_in)
                m2 = jnp.maximum(m, jnp.max(jnp.where(m_in, sid_v, m)))
                r2 = jnp.where(m_in, 0, r)
                return b, m2, r2

            def sv(i, carry2):
                base, maxs = carry2
                off = pl.ds(pl.multiple_of(i * 16, 16), 16)
                sid_v = sid_b[off]
                pot = plsc.load_gather(costs_l, [qs_b[off]]) * rd_b[off]
                ones = jnp.ones((16,), jnp.int32)
                b, m, r = attempt(base, maxs, ones, sid_v, pot)

                def wcond(cr):
                    return jnp.max(cr[2]) > 0

                def wbody(cr):
                    b, m, r = cr
                    flush(b, m)
                    b2 = jnp.min(jnp.where(r == 1, sid_v, POS))
                    return attempt(b2, b2 - 1, r, sid_v, pot)

                b, m, r = lax.while_loop(wcond, wbody, (b, m, r))
                return b, m
            return lax.fori_loop(0, VPS, sv, carry)

        return lax.cond(cmax < base + WIN, fast, slow, (base, maxs))

    def pair(t, carry):
        g = t * 2
        carry = step(g, carry, 0, 1)
        carry = step(g + 1, carry, 1, 0)
        return carry

    base, maxs = lax.fori_loop(
        0, STEPS // 2, pair, (jnp.int32(NEG), jnp.int32(NEG - 1)))
    flush(base, maxs)
    plsc.subcore_barrier()
    pltpu.sync_copy(
        spacc.at[pl.ds(s * SLICE, SLICE)],
        out_hbm.at[c, pl.ds(s * SLICE, SLICE)],
    )


def _sc_call(costs, qs, rd, sid):
    mesh = plsc.VectorSubcoreMesh(core_axis_name="c", subcore_axis_name="s")
    f = pl.kernel(
        _sc_body,
        out_type=jax.ShapeDtypeStruct((2, N_PAD), jnp.float32),
        mesh=mesh,
        scratch_types=[
            pltpu.VMEM((N_SYM,), jnp.float32),
            pltpu.VMEM((2, SUB), jnp.int32),
            pltpu.VMEM((2, SUB), jnp.float32),
            pltpu.VMEM((2, SUB), jnp.int32),
            pltpu.VMEM((WIN,), jnp.float32),
            pltpu.VMEM((FB,), jnp.int32),
            pltpu.VMEM_SHARED((N_PAD,), jnp.float32),
            pltpu.SemaphoreType.DMA((2,)),
            pltpu.SemaphoreType.DMA,
        ],
        compiler_params=pltpu.CompilerParams(
            use_tc_tiling_on_sc=False, needs_layout_passes=False),
    )
    return f(costs, qs, rd, sid)


def _merge_body(p_ref, out_ref):
    out_ref[...] = p_ref[0] + p_ref[1]


def _merge(partials):
    p3 = partials.reshape(2, N_PAD // 128, 128)
    return pl.pallas_call(
        _merge_body,
        out_shape=jax.ShapeDtypeStruct((N_PAD // 128, 128), jnp.float32),
    )(p3)


def kernel(symbol_embeddings, question_symbols, ranking_difference,
           segment_ids, W, b):
    costs = _symbol_costs(symbol_embeddings, W, b).reshape(N_SYM)
    qs = question_symbols.astype(jnp.int32)
    sid = segment_ids.astype(jnp.int32)
    partials = _sc_call(costs, qs, ranking_difference, sid)
    out = _merge(partials).reshape(N_PAD)
    return out[:NUM_SEGMENTS]
